# baseline (speedup 1.0000x reference)
import functools

import jax
import jax.numpy as jnp
from jax import lax
from jax.experimental import pallas as pl
from jax.experimental.pallas import tpu as pltpu
from jax.experimental.pallas import tpu_sc as plsc

N = 10000
E = 320000
D = 128
H = 128
C = 64
F = 128

NC = 2
NS = 16
NW = NC * NS
CHUNK = 128
NCHUNK = 80
NBUF = 2
GROUP = 8
E_PAD = NW * NCHUNK * CHUNK
RPT = 632
NROW = NS * RPT


def _sc_agg_body(h_hbm, src_hbm, dst_hbm, z_hbm, agg_hbm,
                 src_g, dst_v, rows0, rows1,
                 acc, sem0, sem1):
    cid = lax.axis_index("c")
    sid = lax.axis_index("s")
    wid = sid * NC + cid
    rows = (rows0, rows1)
    sems = (sem0, sem1)

    pltpu.sync_copy(z_hbm, acc.at[pl.ds(sid * RPT, RPT)])
    pltpu.sync_copy(dst_hbm.at[wid], dst_v)

    plsc.subcore_barrier()

    def gather(idx, b):
        return pltpu.make_async_copy(h_hbm.at[idx], rows[b], sems[b])

    def group(i, carry):
        c0 = i * GROUP
        pltpu.sync_copy(src_hbm.at[wid, pl.ds(c0, GROUP)], src_g)
        for b in range(NBUF):
            gather(src_g.at[b], b).start()
        for k in range(GROUP):
            gather(src_g.at[k], k % NBUF).wait()
            pltpu.sync_copy(rows[k % NBUF], acc.at[dst_v.at[c0 + k]], add=True)
            if k < GROUP - NBUF:
                gather(src_g.at[k + NBUF], k % NBUF).start()
        return carry

    lax.fori_loop(0, NCHUNK // GROUP, group, 0)

    plsc.subcore_barrier()

    pltpu.sync_copy(acc.at[pl.ds(sid * RPT, RPT)],
                    agg_hbm.at[cid, pl.ds(sid * RPT, RPT)])


_sc_agg = pl.kernel(
    _sc_agg_body,
    out_type=[jax.ShapeDtypeStruct((NC, NROW, F), jnp.float32)],
    mesh=plsc.VectorSubcoreMesh(core_axis_name="c", subcore_axis_name="s"),
    scratch_types=[
        pltpu.VMEM((GROUP, CHUNK), jnp.int32),
        pltpu.VMEM((NCHUNK, CHUNK), jnp.int32),
        pltpu.VMEM((CHUNK, F), jnp.float32),
        pltpu.VMEM((CHUNK, F), jnp.float32),
        pltpu.VMEM_SHARED((NROW, F), jnp.float32),
        pltpu.SemaphoreType.DMA,
        pltpu.SemaphoreType.DMA,
    ],
)


def _sc_deg_body(ones_hbm, dst_hbm, z_hbm, deg_hbm,
                 ones_v, dst_v, acc):
    cid = lax.axis_index("c")
    sid = lax.axis_index("s")
    wid = sid * NC + cid

    pltpu.sync_copy(z_hbm, acc.at[pl.ds(sid * RPT, RPT)])
    pltpu.sync_copy(ones_hbm, ones_v)
    pltpu.sync_copy(dst_hbm.at[wid], dst_v)

    plsc.subcore_barrier()

    def chunk(c, carry):
        pltpu.sync_copy(ones_v, acc.at[dst_v.at[c]], add=True)
        return carry

    lax.fori_loop(0, NCHUNK, chunk, 0)

    plsc.subcore_barrier()

    pltpu.sync_copy(acc.at[pl.ds(sid * RPT, RPT)],
                    deg_hbm.at[cid, pl.ds(sid * RPT, RPT)])


_sc_deg = pl.kernel(
    _sc_deg_body,
    out_type=[jax.ShapeDtypeStruct((NC, NROW, F), jnp.float32)],
    mesh=plsc.VectorSubcoreMesh(core_axis_name="c", subcore_axis_name="s"),
    scratch_types=[
        pltpu.VMEM((CHUNK, F), jnp.float32),
        pltpu.VMEM((NCHUNK, CHUNK), jnp.int32),
        pltpu.VMEM_SHARED((NROW, F), jnp.float32),
    ],
)


def _tc1_body(aggp, degp, x, wl, wr, b, g, be, out, dinv_out):
    agg = aggp[0, :N] + aggp[1, :N]
    deg = degp[0, :N, 0:1] + degp[1, :N, 0:1]
    dinv = 1.0 / jnp.maximum(deg, 1.0)
    h = (jnp.dot(agg * dinv, wl[...], preferred_element_type=jnp.float32)
         + jnp.dot(x[...], wr[...], preferred_element_type=jnp.float32)
         + b[...])
    m = jnp.mean(h, axis=0)
    v = jnp.mean((h - m) ** 2, axis=0)
    hn = (h - m) * lax.rsqrt(v + 1e-5) * g[...] + be[...]
    out[...] = jnp.maximum(hn, 0.0)
    dinv_out[...] = dinv


def _tc2_body(aggp, dinv, h1, wl, wr, b, g, be, wl2, wr2, b2, p2, r2):
    agg = (aggp[0, :N] + aggp[1, :N]) * dinv[...]
    h = (jnp.dot(agg, wl[...], preferred_element_type=jnp.float32)
         + jnp.dot(h1[...], wr[...], preferred_element_type=jnp.float32)
         + b[...])
    m = jnp.mean(h, axis=0)
    v = jnp.mean((h - m) ** 2, axis=0)
    hn = (h - m) * lax.rsqrt(v + 1e-5) * g[...] + be[...]
    h2 = jnp.maximum(hn, 0.0)
    p2[...] = jnp.dot(h2, wl2[...], preferred_element_type=jnp.float32)
    r2[...] = jnp.dot(h2, wr2[...], preferred_element_type=jnp.float32) + b2[...]


def _tc3_body(aggp, dinv, r2, out):
    z = (aggp[0, :N, :C] + aggp[1, :N, :C]) * dinv[...] + r2[...]
    m = jnp.max(z, axis=1, keepdims=True)
    e = jnp.exp(z - m)
    s = jnp.sum(e, axis=1, keepdims=True)
    out[...] = z - m - jnp.log(s)


def kernel(x, edge_index, W_l0, W_r0, b0, g0, be0,
           W_l1, W_r1, b1, g1, be1, W_l2, W_r2, b2):
    src = edge_index[0]
    dst = edge_index[1]
    pad = E_PAD - E
    src_p = jnp.concatenate([src, jnp.zeros((pad,), jnp.int32)])
    dst_p = jnp.concatenate([dst, jnp.full((pad,), N, jnp.int32)])
    src_r = src_p.reshape(NW, NCHUNK, CHUNK)
    dst_r = dst_p.reshape(NW, NCHUNK, CHUNK)
    z = jnp.zeros((RPT, F), jnp.float32)
    ones_t = jnp.ones((CHUNK, F), jnp.float32)
    wl2p = jnp.concatenate([W_l2, jnp.zeros((H, F - C), jnp.float32)], axis=1)

    b0r, g0r, be0r = b0[None, :], g0[None, :], be0[None, :]
    b1r, g1r, be1r = b1[None, :], g1[None, :], be1[None, :]
    b2r = b2[None, :]

    degp, = _sc_deg(ones_t, dst_r, z)
    aggp0, = _sc_agg(x, src_r, dst_r, z)

    h1, dinv = pl.pallas_call(
        _tc1_body,
        out_shape=(jax.ShapeDtypeStruct((N, H), jnp.float32),
                   jax.ShapeDtypeStruct((N, 1), jnp.float32)),
    )(aggp0, degp, x, W_l0, W_r0, b0r, g0r, be0r)

    aggp1, = _sc_agg(h1, src_r, dst_r, z)

    p2, r2 = pl.pallas_call(
        _tc2_body,
        out_shape=(jax.ShapeDtypeStruct((N, F), jnp.float32),
                   jax.ShapeDtypeStruct((N, C), jnp.float32)),
    )(aggp1, dinv, h1, W_l1, W_r1, b1r, g1r, be1r, wl2p, W_r2, b2r)

    aggp2, = _sc_agg(p2, src_r, dst_r, z)

    out = pl.pallas_call(
        _tc3_body,
        out_shape=jax.ShapeDtypeStruct((N, C), jnp.float32),
    )(aggp2, dinv, r2)

    return out


# speedup vs baseline: 2.9479x; 2.9479x over previous
import functools

import jax
import jax.numpy as jnp
from jax import lax
from jax.experimental import pallas as pl
from jax.experimental.pallas import tpu as pltpu
from jax.experimental.pallas import tpu_sc as plsc

N = 10000
E = 320000
D = 128
H = 128
C = 64
F = 128

NC = 2
NS = 16
NW = NC * NS
CHUNK = 128
NCHUNK = 80
NBUF = 2
GROUP = 8
E_PAD = NW * NCHUNK * CHUNK
RPT = 632
NROW = NS * RPT


def _sc_agg_body(h_hbm, src_hbm, dst_hbm, z_hbm, agg_hbm,
                 src_g, dst_v, rows0, rows1,
                 acc, sem0, sem1):
    cid = lax.axis_index("c")
    sid = lax.axis_index("s")
    wid = sid * NC + cid
    rows = (rows0, rows1)
    sems = (sem0, sem1)

    pltpu.sync_copy(z_hbm, acc.at[pl.ds(sid * RPT, RPT)])
    pltpu.sync_copy(dst_hbm.at[wid], dst_v)

    plsc.subcore_barrier()

    def gather(idx, b):
        return pltpu.make_async_copy(h_hbm.at[idx], rows[b], sems[b])

    def group(i, carry):
        c0 = i * GROUP
        pltpu.sync_copy(src_hbm.at[wid, pl.ds(c0, GROUP)], src_g)
        for b in range(NBUF):
            gather(src_g.at[b], b).start()
        for k in range(GROUP):
            gather(src_g.at[k], k % NBUF).wait()
            pltpu.sync_copy(rows[k % NBUF], acc.at[dst_v.at[c0 + k]], add=True)
            if k < GROUP - NBUF:
                gather(src_g.at[k + NBUF], k % NBUF).start()
        return carry

    lax.fori_loop(0, NCHUNK // GROUP, group, 0)

    plsc.subcore_barrier()

    pltpu.sync_copy(acc.at[pl.ds(sid * RPT, RPT)],
                    agg_hbm.at[cid, pl.ds(sid * RPT, RPT)])


_sc_agg = pl.kernel(
    _sc_agg_body,
    out_type=[jax.ShapeDtypeStruct((NC, NROW, F), jnp.float32)],
    mesh=plsc.VectorSubcoreMesh(core_axis_name="c", subcore_axis_name="s"),
    scratch_types=[
        pltpu.VMEM((GROUP, CHUNK), jnp.int32),
        pltpu.VMEM((NCHUNK, CHUNK), jnp.int32),
        pltpu.VMEM((CHUNK, F), jnp.float32),
        pltpu.VMEM((CHUNK, F), jnp.float32),
        pltpu.VMEM_SHARED((NROW, F), jnp.float32),
        pltpu.SemaphoreType.DMA,
        pltpu.SemaphoreType.DMA,
    ],
)


def _sc_deg_body(ones_hbm, dst_hbm, z_hbm, deg_hbm,
                 ones_v, dst_v, acc):
    cid = lax.axis_index("c")
    sid = lax.axis_index("s")
    wid = sid * NC + cid

    pltpu.sync_copy(z_hbm, acc.at[pl.ds(sid * RPT, RPT)])
    pltpu.sync_copy(ones_hbm, ones_v)
    pltpu.sync_copy(dst_hbm.at[wid], dst_v)

    plsc.subcore_barrier()

    def chunk(c, carry):
        pltpu.sync_copy(ones_v, acc.at[dst_v.at[c]], add=True)
        return carry

    lax.fori_loop(0, NCHUNK, chunk, 0)

    plsc.subcore_barrier()

    pltpu.sync_copy(acc.at[pl.ds(sid * RPT, RPT)],
                    deg_hbm.at[cid, pl.ds(sid * RPT, RPT)])


_sc_deg = pl.kernel(
    _sc_deg_body,
    out_type=[jax.ShapeDtypeStruct((NC, NROW, F), jnp.float32)],
    mesh=plsc.VectorSubcoreMesh(core_axis_name="c", subcore_axis_name="s"),
    scratch_types=[
        pltpu.VMEM((CHUNK, F), jnp.float32),
        pltpu.VMEM((NCHUNK, CHUNK), jnp.int32),
        pltpu.VMEM_SHARED((NROW, F), jnp.float32),
    ],
)


def _tc1_body(aggp, degp, x, wl, wr, b, g, be, out, dinv_out):
    agg = aggp[0, :N] + aggp[1, :N]
    deg = degp[0, :N, 0:1] + degp[1, :N, 0:1]
    dinv = 1.0 / jnp.maximum(deg, 1.0)
    h = (jnp.dot(agg * dinv, wl[...], preferred_element_type=jnp.float32)
         + jnp.dot(x[...], wr[...], preferred_element_type=jnp.float32)
         + b[...])
    m = jnp.mean(h, axis=0)
    v = jnp.mean((h - m) ** 2, axis=0)
    hn = (h - m) * lax.rsqrt(v + 1e-5) * g[...] + be[...]
    out[...] = jnp.maximum(hn, 0.0)
    dinv_out[...] = dinv


def _tc2_body(aggp, dinv, h1, wl, wr, b, g, be, wl2, wr2, b2, p2, r2):
    agg = (aggp[0, :N] + aggp[1, :N]) * dinv[...]
    h = (jnp.dot(agg, wl[...], preferred_element_type=jnp.float32)
         + jnp.dot(h1[...], wr[...], preferred_element_type=jnp.float32)
         + b[...])
    m = jnp.mean(h, axis=0)
    v = jnp.mean((h - m) ** 2, axis=0)
    hn = (h - m) * lax.rsqrt(v + 1e-5) * g[...] + be[...]
    h2 = jnp.maximum(hn, 0.0)
    p2[...] = jnp.dot(h2, wl2[...], preferred_element_type=jnp.float32)
    r2[...] = jnp.dot(h2, wr2[...], preferred_element_type=jnp.float32) + b2[...]


def _tc3_body(aggp, dinv, r2, out):
    z = (aggp[0, :N, :C] + aggp[1, :N, :C]) * dinv[...] + r2[...]
    m = jnp.max(z, axis=1, keepdims=True)
    e = jnp.exp(z - m)
    s = jnp.sum(e, axis=1, keepdims=True)
    out[...] = z - m - jnp.log(s)


def kernel(x, edge_index, W_l0, W_r0, b0, g0, be0,
           W_l1, W_r1, b1, g1, be1, W_l2, W_r2, b2):
    src = edge_index[0]
    dst = edge_index[1]
    pad = E_PAD - E
    pad_i = jnp.arange(pad, dtype=jnp.int32)
    src_p = jnp.concatenate([src, pad_i % N])
    dst_p = jnp.concatenate([dst, N + pad_i % (NROW - N)])
    src_r = src_p.reshape(NW, NCHUNK, CHUNK)
    dst_r = dst_p.reshape(NW, NCHUNK, CHUNK)
    z = jnp.zeros((RPT, F), jnp.float32)
    ones_t = jnp.ones((CHUNK, F), jnp.float32)
    wl2p = jnp.concatenate([W_l2, jnp.zeros((H, F - C), jnp.float32)], axis=1)

    b0r, g0r, be0r = b0[None, :], g0[None, :], be0[None, :]
    b1r, g1r, be1r = b1[None, :], g1[None, :], be1[None, :]
    b2r = b2[None, :]

    degp, = _sc_deg(ones_t, dst_r, z)
    aggp0, = _sc_agg(x, src_r, dst_r, z)

    h1, dinv = pl.pallas_call(
        _tc1_body,
        out_shape=(jax.ShapeDtypeStruct((N, H), jnp.float32),
                   jax.ShapeDtypeStruct((N, 1), jnp.float32)),
    )(aggp0, degp, x, W_l0, W_r0, b0r, g0r, be0r)

    aggp1, = _sc_agg(h1, src_r, dst_r, z)

    p2, r2 = pl.pallas_call(
        _tc2_body,
        out_shape=(jax.ShapeDtypeStruct((N, F), jnp.float32),
                   jax.ShapeDtypeStruct((N, C), jnp.float32)),
    )(aggp1, dinv, h1, W_l1, W_r1, b1r, g1r, be1r, wl2p, W_r2, b2r)

    aggp2, = _sc_agg(p2, src_r, dst_r, z)

    out = pl.pallas_call(
        _tc3_body,
        out_shape=jax.ShapeDtypeStruct((N, C), jnp.float32),
    )(aggp2, dinv, r2)

    return out


# speedup vs baseline: 2.9542x; 1.0021x over previous
import functools

import jax
import jax.numpy as jnp
from jax import lax
from jax.experimental import pallas as pl
from jax.experimental.pallas import tpu as pltpu
from jax.experimental.pallas import tpu_sc as plsc

N = 10000
E = 320000
D = 128
H = 128
C = 64
F = 128

NC = 2
NS = 16
NW = NC * NS
CHUNK = 128
NCHUNK = 80
NBUF = 2
GROUP = 8
E_PAD = NW * NCHUNK * CHUNK
RPT = 632
NROW = NS * RPT


def _sc_agg_body(h_hbm, src_hbm, dst_hbm, z_hbm, agg_hbm,
                 src_g, dst_v, rows0, rows1,
                 acc, sem0, sem1):
    cid = lax.axis_index("c")
    sid = lax.axis_index("s")
    wid = sid * NC + cid
    rows = (rows0, rows1)
    sems = (sem0, sem1)

    pltpu.sync_copy(z_hbm, acc.at[pl.ds(sid * RPT, RPT)])
    pltpu.sync_copy(dst_hbm.at[wid], dst_v)

    plsc.subcore_barrier()

    def gather(idx, b):
        return pltpu.make_async_copy(h_hbm.at[idx], rows[b], sems[b])

    def group(i, carry):
        c0 = i * GROUP
        pltpu.sync_copy(src_hbm.at[wid, pl.ds(c0, GROUP)], src_g)
        for b in range(NBUF):
            gather(src_g.at[b], b).start()
        for k in range(GROUP):
            gather(src_g.at[k], k % NBUF).wait()
            pltpu.sync_copy(rows[k % NBUF], acc.at[dst_v.at[c0 + k]], add=True)
            if k < GROUP - NBUF:
                gather(src_g.at[k + NBUF], k % NBUF).start()
        return carry

    lax.fori_loop(0, NCHUNK // GROUP, group, 0)

    plsc.subcore_barrier()

    pltpu.sync_copy(acc.at[pl.ds(sid * RPT, RPT)],
                    agg_hbm.at[cid, pl.ds(sid * RPT, RPT)])


_sc_agg = pl.kernel(
    _sc_agg_body,
    out_type=[jax.ShapeDtypeStruct((NC, NROW, F), jnp.float32)],
    mesh=plsc.VectorSubcoreMesh(core_axis_name="c", subcore_axis_name="s"),
    scratch_types=[
        pltpu.VMEM((GROUP, CHUNK), jnp.int32),
        pltpu.VMEM((NCHUNK, CHUNK), jnp.int32),
        pltpu.VMEM((CHUNK, F), jnp.float32),
        pltpu.VMEM((CHUNK, F), jnp.float32),
        pltpu.VMEM_SHARED((NROW, F), jnp.float32),
        pltpu.SemaphoreType.DMA,
        pltpu.SemaphoreType.DMA,
    ],
)


DW = 128


def _sc_deg_body(ones_hbm, dst_hbm, z8_hbm, deg_hbm,
                 ones_v, dst_v, acc2):
    cid = lax.axis_index("c")
    sid = lax.axis_index("s")
    wid = sid * NC + cid

    pltpu.sync_copy(z8_hbm, acc2.at[pl.ds(sid * RPT, RPT)])
    pltpu.sync_copy(ones_hbm, ones_v)
    pltpu.sync_copy(dst_hbm.at[wid], dst_v)

    plsc.subcore_barrier()

    def chunk(c, carry):
        pltpu.sync_copy(ones_v, acc2.at[dst_v.at[c]], add=True)
        return carry

    lax.fori_loop(0, NCHUNK, chunk, 0)

    plsc.subcore_barrier()

    pltpu.sync_copy(acc2.at[pl.ds(sid * RPT, RPT)],
                    deg_hbm.at[cid, pl.ds(sid * RPT, RPT)])


_sc_deg = pl.kernel(
    _sc_deg_body,
    out_type=[jax.ShapeDtypeStruct((NC, NROW, DW), jnp.float32)],
    mesh=plsc.VectorSubcoreMesh(core_axis_name="c", subcore_axis_name="s"),
    scratch_types=[
        pltpu.VMEM((CHUNK, DW), jnp.float32),
        pltpu.VMEM((NCHUNK, CHUNK), jnp.int32),
        pltpu.VMEM_SHARED((NROW, DW), jnp.float32),
    ],
)


def _tc1_body(aggp, degp, x, wl, wr, b, g, be, out, dinv_out):
    agg = aggp[0, :N] + aggp[1, :N]
    deg = degp[0, :N, 0:1] + degp[1, :N, 0:1]
    dinv = 1.0 / jnp.maximum(deg, 1.0)
    h = (jnp.dot(agg * dinv, wl[...], preferred_element_type=jnp.float32)
         + jnp.dot(x[...], wr[...], preferred_element_type=jnp.float32)
         + b[...])
    m = jnp.mean(h, axis=0)
    v = jnp.mean((h - m) ** 2, axis=0)
    hn = (h - m) * lax.rsqrt(v + 1e-5) * g[...] + be[...]
    out[...] = jnp.maximum(hn, 0.0)
    dinv_out[...] = dinv


def _tc2_body(aggp, dinv, h1, wl, wr, b, g, be, wl2, wr2, b2, p2, r2):
    agg = (aggp[0, :N] + aggp[1, :N]) * dinv[...]
    h = (jnp.dot(agg, wl[...], preferred_element_type=jnp.float32)
         + jnp.dot(h1[...], wr[...], preferred_element_type=jnp.float32)
         + b[...])
    m = jnp.mean(h, axis=0)
    v = jnp.mean((h - m) ** 2, axis=0)
    hn = (h - m) * lax.rsqrt(v + 1e-5) * g[...] + be[...]
    h2 = jnp.maximum(hn, 0.0)
    p2[...] = jnp.dot(h2, wl2[...], preferred_element_type=jnp.float32)
    r2[...] = jnp.dot(h2, wr2[...], preferred_element_type=jnp.float32) + b2[...]


def _tc3_body(aggp, dinv, r2, out):
    z = (aggp[0, :N, :C] + aggp[1, :N, :C]) * dinv[...] + r2[...]
    m = jnp.max(z, axis=1, keepdims=True)
    e = jnp.exp(z - m)
    s = jnp.sum(e, axis=1, keepdims=True)
    out[...] = z - m - jnp.log(s)


def kernel(x, edge_index, W_l0, W_r0, b0, g0, be0,
           W_l1, W_r1, b1, g1, be1, W_l2, W_r2, b2):
    src = edge_index[0]
    dst = edge_index[1]
    pad = E_PAD - E
    pad_i = jnp.arange(pad, dtype=jnp.int32)
    src_p = jnp.concatenate([src, pad_i % N])
    dst_p = jnp.concatenate([dst, N + pad_i % (NROW - N)])
    src_r = src_p.reshape(NW, NCHUNK, CHUNK)
    dst_r = dst_p.reshape(NW, NCHUNK, CHUNK)
    z = jnp.zeros((RPT, F), jnp.float32)
    wl2p = jnp.concatenate([W_l2, jnp.zeros((H, F - C), jnp.float32)], axis=1)

    b0r, g0r, be0r = b0[None, :], g0[None, :], be0[None, :]
    b1r, g1r, be1r = b1[None, :], g1[None, :], be1[None, :]
    b2r = b2[None, :]

    z8 = jnp.zeros((RPT, DW), jnp.float32)
    ones8 = jnp.ones((CHUNK, DW), jnp.float32)
    degp, = _sc_deg(ones8, dst_r, z8)
    aggp0, = _sc_agg(x, src_r, dst_r, z)

    h1, dinv = pl.pallas_call(
        _tc1_body,
        out_shape=(jax.ShapeDtypeStruct((N, H), jnp.float32),
                   jax.ShapeDtypeStruct((N, 1), jnp.float32)),
    )(aggp0, degp, x, W_l0, W_r0, b0r, g0r, be0r)

    aggp1, = _sc_agg(h1, src_r, dst_r, z)

    p2, r2 = pl.pallas_call(
        _tc2_body,
        out_shape=(jax.ShapeDtypeStruct((N, F), jnp.float32),
                   jax.ShapeDtypeStruct((N, C), jnp.float32)),
    )(aggp1, dinv, h1, W_l1, W_r1, b1r, g1r, be1r, wl2p, W_r2, b2r)

    aggp2, = _sc_agg(p2, src_r, dst_r, z)

    out = pl.pallas_call(
        _tc3_body,
        out_shape=jax.ShapeDtypeStruct((N, C), jnp.float32),
    )(aggp2, dinv, r2)

    return out


# speedup vs baseline: 3.0429x; 1.0300x over previous
import functools

import jax
import jax.numpy as jnp
from jax import lax
from jax.experimental import pallas as pl
from jax.experimental.pallas import tpu as pltpu
from jax.experimental.pallas import tpu_sc as plsc

N = 10000
E = 320000
D = 128
H = 128
C = 64
F = 128

NC = 2
NS = 16
NW = NC * NS
CHUNK = 96
NCHUNK = 112
NBUF = 3
GROUP = 16
E_PAD = NW * NCHUNK * CHUNK
RPT = 632
NROW = NS * RPT


def _sc_agg_body(h_hbm, src_hbm, dst_hbm, z_hbm, agg_hbm,
                 src_g, dst_v, rows0, rows1, rows2,
                 acc, sem0, sem1, sem2):
    cid = lax.axis_index("c")
    sid = lax.axis_index("s")
    wid = sid * NC + cid
    rows = (rows0, rows1, rows2)
    sems = (sem0, sem1, sem2)

    pltpu.sync_copy(z_hbm, acc.at[pl.ds(sid * RPT, RPT)])

    plsc.subcore_barrier()

    def gather(idx, b):
        return pltpu.make_async_copy(h_hbm.at[idx], rows[b], sems[b])

    def group(i, carry):
        c0 = i * GROUP
        pltpu.sync_copy(src_hbm.at[wid, pl.ds(c0, GROUP)], src_g)
        pltpu.sync_copy(dst_hbm.at[wid, pl.ds(c0, GROUP)], dst_v)
        for b in range(NBUF):
            gather(src_g.at[b], b).start()
        for k in range(GROUP):
            gather(src_g.at[k], k % NBUF).wait()
            pltpu.sync_copy(rows[k % NBUF], acc.at[dst_v.at[k]], add=True)
            if k < GROUP - NBUF:
                gather(src_g.at[k + NBUF], k % NBUF).start()
        return carry

    lax.fori_loop(0, NCHUNK // GROUP, group, 0)

    plsc.subcore_barrier()

    pltpu.sync_copy(acc.at[pl.ds(sid * RPT, RPT)],
                    agg_hbm.at[cid, pl.ds(sid * RPT, RPT)])


_sc_agg = pl.kernel(
    _sc_agg_body,
    out_type=[jax.ShapeDtypeStruct((NC, NROW, F), jnp.float32)],
    mesh=plsc.VectorSubcoreMesh(core_axis_name="c", subcore_axis_name="s"),
    scratch_types=[
        pltpu.VMEM((GROUP, CHUNK), jnp.int32),
        pltpu.VMEM((GROUP, CHUNK), jnp.int32),
        pltpu.VMEM((CHUNK, F), jnp.float32),
        pltpu.VMEM((CHUNK, F), jnp.float32),
        pltpu.VMEM((CHUNK, F), jnp.float32),
        pltpu.VMEM_SHARED((NROW, F), jnp.float32),
        pltpu.SemaphoreType.DMA,
        pltpu.SemaphoreType.DMA,
        pltpu.SemaphoreType.DMA,
    ],
)


DW = 128


def _sc_deg_body(ones_hbm, dst_hbm, z8_hbm, deg_hbm,
                 ones_v, dst_v, acc2):
    cid = lax.axis_index("c")
    sid = lax.axis_index("s")
    wid = sid * NC + cid

    pltpu.sync_copy(z8_hbm, acc2.at[pl.ds(sid * RPT, RPT)])
    pltpu.sync_copy(ones_hbm, ones_v)
    pltpu.sync_copy(dst_hbm.at[wid], dst_v)

    plsc.subcore_barrier()

    def chunk(c, carry):
        pltpu.sync_copy(ones_v, acc2.at[dst_v.at[c]], add=True)
        return carry

    lax.fori_loop(0, NCHUNK, chunk, 0)

    plsc.subcore_barrier()

    pltpu.sync_copy(acc2.at[pl.ds(sid * RPT, RPT)],
                    deg_hbm.at[cid, pl.ds(sid * RPT, RPT)])


_sc_deg = pl.kernel(
    _sc_deg_body,
    out_type=[jax.ShapeDtypeStruct((NC, NROW, DW), jnp.float32)],
    mesh=plsc.VectorSubcoreMesh(core_axis_name="c", subcore_axis_name="s"),
    scratch_types=[
        pltpu.VMEM((CHUNK, DW), jnp.float32),
        pltpu.VMEM((NCHUNK, CHUNK), jnp.int32),
        pltpu.VMEM_SHARED((NROW, DW), jnp.float32),
    ],
)


def _tc1_body(aggp, degp, x, wl, wr, b, g, be, out, dinv_out):
    agg = aggp[0, :N] + aggp[1, :N]
    deg = degp[0, :N, 0:1] + degp[1, :N, 0:1]
    dinv = 1.0 / jnp.maximum(deg, 1.0)
    h = (jnp.dot(agg * dinv, wl[...], preferred_element_type=jnp.float32)
         + jnp.dot(x[...], wr[...], preferred_element_type=jnp.float32)
         + b[...])
    m = jnp.mean(h, axis=0)
    v = jnp.mean((h - m) ** 2, axis=0)
    hn = (h - m) * lax.rsqrt(v + 1e-5) * g[...] + be[...]
    out[...] = jnp.maximum(hn, 0.0)
    dinv_out[...] = dinv


def _tc2_body(aggp, dinv, h1, wl, wr, b, g, be, wl2, wr2, b2, p2, r2):
    agg = (aggp[0, :N] + aggp[1, :N]) * dinv[...]
    h = (jnp.dot(agg, wl[...], preferred_element_type=jnp.float32)
         + jnp.dot(h1[...], wr[...], preferred_element_type=jnp.float32)
         + b[...])
    m = jnp.mean(h, axis=0)
    v = jnp.mean((h - m) ** 2, axis=0)
    hn = (h - m) * lax.rsqrt(v + 1e-5) * g[...] + be[...]
    h2 = jnp.maximum(hn, 0.0)
    p2[...] = jnp.dot(h2, wl2[...], preferred_element_type=jnp.float32)
    r2[...] = jnp.dot(h2, wr2[...], preferred_element_type=jnp.float32) + b2[...]


def _tc3_body(aggp, dinv, r2, out):
    z = (aggp[0, :N, :C] + aggp[1, :N, :C]) * dinv[...] + r2[...]
    m = jnp.max(z, axis=1, keepdims=True)
    e = jnp.exp(z - m)
    s = jnp.sum(e, axis=1, keepdims=True)
    out[...] = z - m - jnp.log(s)


def kernel(x, edge_index, W_l0, W_r0, b0, g0, be0,
           W_l1, W_r1, b1, g1, be1, W_l2, W_r2, b2):
    src = edge_index[0]
    dst = edge_index[1]
    pad = E_PAD - E
    pad_i = jnp.arange(pad, dtype=jnp.int32)
    src_p = jnp.concatenate([src, pad_i % N])
    dst_p = jnp.concatenate([dst, N + pad_i % (NROW - N)])
    src_r = src_p.reshape(NW, NCHUNK, CHUNK)
    dst_r = dst_p.reshape(NW, NCHUNK, CHUNK)
    z = jnp.zeros((RPT, F), jnp.float32)
    wl2p = jnp.concatenate([W_l2, jnp.zeros((H, F - C), jnp.float32)], axis=1)

    b0r, g0r, be0r = b0[None, :], g0[None, :], be0[None, :]
    b1r, g1r, be1r = b1[None, :], g1[None, :], be1[None, :]
    b2r = b2[None, :]

    z8 = jnp.zeros((RPT, DW), jnp.float32)
    ones8 = jnp.ones((CHUNK, DW), jnp.float32)
    degp, = _sc_deg(ones8, dst_r, z8)
    aggp0, = _sc_agg(x, src_r, dst_r, z)

    h1, dinv = pl.pallas_call(
        _tc1_body,
        out_shape=(jax.ShapeDtypeStruct((N, H), jnp.float32),
                   jax.ShapeDtypeStruct((N, 1), jnp.float32)),
    )(aggp0, degp, x, W_l0, W_r0, b0r, g0r, be0r)

    aggp1, = _sc_agg(h1, src_r, dst_r, z)

    p2, r2 = pl.pallas_call(
        _tc2_body,
        out_shape=(jax.ShapeDtypeStruct((N, F), jnp.float32),
                   jax.ShapeDtypeStruct((N, C), jnp.float32)),
    )(aggp1, dinv, h1, W_l1, W_r1, b1r, g1r, be1r, wl2p, W_r2, b2r)

    aggp2, = _sc_agg(p2, src_r, dst_r, z)

    out = pl.pallas_call(
        _tc3_body,
        out_shape=jax.ShapeDtypeStruct((N, C), jnp.float32),
    )(aggp2, dinv, r2)

    return out
